# baseline (speedup 1.0000x reference)
import functools

import jax
import jax.numpy as jnp
from jax import lax
from jax.experimental import pallas as pl
from jax.experimental.pallas import tpu as pltpu
from jax.experimental.pallas import tpu_sc as plsc

H = 8
LANES = 128


GROUP = 16


def _transform_body(x_ref, m_ref, s_ref, b_ref, o_ref):
    x = x_ref[...]
    y = jnp.dot(x, m_ref[...], preferred_element_type=jnp.float32)
    o_ref[...] = y * s_ref[0, 0] + b_ref[...]


def _transform_table(table, rotation_matrix, group_scale, group_bias):
    V = table.shape[0]
    rows = V // GROUP
    rows_pad = (rows + 7) // 8 * 8
    x = table.reshape(rows, GROUP * H)
    big_r = jnp.kron(jnp.eye(GROUP, dtype=table.dtype), rotation_matrix)
    bias_row = jnp.tile(group_bias, GROUP).reshape(1, GROUP * H)
    scale = group_scale.reshape(1, 1)
    blk = 4096
    grid = (rows_pad + blk - 1) // blk
    out = pl.pallas_call(
        _transform_body,
        grid=(grid,),
        in_specs=[
            pl.BlockSpec((blk, GROUP * H), lambda i: (i, 0)),
            pl.BlockSpec((GROUP * H, GROUP * H), lambda i: (0, 0)),
            pl.BlockSpec((1, 1), lambda i: (0, 0)),
            pl.BlockSpec((1, GROUP * H), lambda i: (0, 0)),
        ],
        out_specs=pl.BlockSpec((blk, GROUP * H), lambda i: (i, 0)),
        out_shape=jax.ShapeDtypeStruct((rows_pad, GROUP * H), jnp.float32),
    )(x, big_r, scale, bias_row)
    return out.reshape(rows_pad * GROUP, H)


def _make_gather(N, V):
    info = plsc.get_sparse_core_info()
    NC, NS = info.num_cores, info.num_subcores
    NW = NC * NS
    per_w = N // NW
    C = 4096
    n_chunks = per_w // C
    mesh = plsc.VectorSubcoreMesh(core_axis_name="c", subcore_axis_name="s")

    @functools.partial(
        pl.kernel,
        out_type=jax.ShapeDtypeStruct((N, H), jnp.float32),
        mesh=mesh,
        compiler_params=pltpu.CompilerParams(use_tc_tiling_on_sc=False),
        scratch_types=[
            pltpu.VMEM((2, C), jnp.int32),
            pltpu.VMEM((2, C, H), jnp.float32),
            pltpu.SemaphoreType.DMA,
            pltpu.SemaphoreType.DMA,
            pltpu.SemaphoreType.DMA,
            pltpu.SemaphoreType.DMA,
            pltpu.SemaphoreType.DMA,
            pltpu.SemaphoreType.DMA,
        ],
    )
    def gather_kernel(
        ids_hbm, tbl_hbm, out_hbm, idx_v, rows_v,
        isem0, isem1, gsem0, gsem1, wsem0, wsem1,
    ):
        isems = (isem0, isem1)
        gsems = (gsem0, gsem1)
        wsems = (wsem0, wsem1)
        wid = lax.axis_index("s") * NC + lax.axis_index("c")
        base = wid * per_w

        def ids_cp(c, p):
            return pltpu.make_async_copy(
                ids_hbm.at[pl.ds(base + c * C, C)], idx_v.at[p], isems[p]
            )

        def gather_cp(p):
            return pltpu.make_async_copy(
                tbl_hbm.at[idx_v.at[p]], rows_v.at[p], gsems[p]
            )

        UPC = C // LANES

        def write_cp(c, p):
            cps = []
            r0 = base // LANES + c * UPC
            for j in range(UPC):
                r = r0 + j
                s8 = r // (LANES * H)
                rem = r % (LANES * H)
                b128 = rem // H
                sm = rem % H
                t = (s8 * H + sm) * LANES + b128
                cps.append(pltpu.make_async_copy(
                    rows_v.at[p, pl.ds(j * LANES, LANES)],
                    out_hbm.at[pl.ds(t * LANES, LANES)],
                    wsems[p],
                ))
            return cps

        ids_cp(0, 0).start()
        ids_cp(0, 0).wait()
        gather_cp(0).start()
        ids_cp(1, 1).start()

        def step(c, p):
            q = 1 - p
            gather_cp(p).wait()
            for cp in write_cp(c, p):
                cp.start()

            def drain_writes():
                for cp in write_cp(c - 1, q):
                    cp.wait()

            pl.when(c >= 1)(drain_writes)

            def advance():
                ids_cp(c + 1, q).wait()
                gather_cp(q).start()

            pl.when(c + 1 < n_chunks)(advance)
            pl.when(c + 2 < n_chunks)(lambda: ids_cp(c + 2, p).start())

        def body(k, carry):
            step(k * 2, 0)
            step(k * 2 + 1, 1)
            return carry

        lax.fori_loop(0, n_chunks // 2, body, 0)
        if n_chunks % 2:
            step(n_chunks - 1, 0)
        for cp in write_cp(n_chunks - 1, (n_chunks - 1) % 2):
            cp.wait()

    return gather_kernel


def _transpose_body(x_ref, o_ref):
    blk_units = x_ref.shape[0] // H
    x = x_ref[...].reshape(blk_units * LANES, H)
    x = x.reshape(blk_units, LANES, H)
    o_ref[...] = x.transpose(0, 2, 1).reshape(blk_units * H, LANES)


def _tile_transpose(flat):
    R = flat.shape[0]
    blk = 1024
    grid = R // blk
    return pl.pallas_call(
        _transpose_body,
        grid=(grid,),
        in_specs=[pl.BlockSpec((blk, LANES), lambda i: (i, 0))],
        out_specs=pl.BlockSpec((blk, LANES), lambda i: (i, 0)),
        out_shape=jax.ShapeDtypeStruct((R, LANES), jnp.float32),
    )(flat)


def kernel(input_ids, table, rotation_matrix, group_scale, group_bias):
    B, S = input_ids.shape
    V = table.shape[0]
    nb = B // LANES
    n_tiles = S * nb
    N = B * S

    t_prime = _transform_table(table, rotation_matrix, group_scale, group_bias)

    ids_v = (
        input_ids.astype(jnp.int32)
        .T.reshape(S // H, H, nb, LANES)
        .transpose(0, 2, 1, 3)
        .reshape(N)
    )

    rows = _make_gather(N, t_prime.shape[0])(ids_v, t_prime)
    return (
        rows.reshape(S, nb, LANES, H)
        .transpose(1, 2, 0, 3)
        .reshape(B, S, H)
    )
